# baseline (speedup 1.0000x reference)
import functools
import math

import jax
import jax.numpy as jnp
from jax import lax
from jax.experimental import pallas as pl
from jax.experimental.pallas import tpu as pltpu
from jax.experimental.pallas import tpu_sc as plsc

D_MODEL = 64
LANES = 16
NUM_CORES = 2
NUM_SUBCORES = 16
NUM_WORKERS = NUM_CORES * NUM_SUBCORES
CHUNK = 128
NBUF = 4


def _emb_body(n_chunks, scale, x_hbm, table_hbm, out_hbm, idx_v, raw_v, scl_v,
              gsem, osem):
  cid = lax.axis_index("c")
  sid = lax.axis_index("s")
  wid = sid * NUM_CORES + cid

  pltpu.sync_copy(x_hbm.at[wid], idx_v)

  def gather_start(c, b):
    pltpu.async_copy(table_hbm.at[idx_v.at[c]], raw_v.at[b], gsem.at[b])

  def gather_wait(b):
    pltpu.make_async_copy(table_hbm.at[idx_v.at[0]], raw_v.at[b],
                          gsem.at[b]).wait()

  def out_start(c, b):
    base = (wid * n_chunks + c) * CHUNK
    pltpu.async_copy(scl_v.at[b], out_hbm.at[pl.ds(base, CHUNK)], osem.at[b])

  def out_wait(b):
    pltpu.make_async_copy(scl_v.at[b], out_hbm.at[pl.ds(0, CHUNK)],
                          osem.at[b]).wait()

  for b in range(NBUF):
    gather_start(jnp.int32(b), b)

  def group(g, carry):
    c0 = g * NBUF
    for b in range(NBUF):
      c = c0 + b
      gather_wait(b)

      @pl.when(c >= NBUF)
      def _():
        out_wait(b)

      @plsc.parallel_loop(0, CHUNK, unroll=8)
      def _(r):
        for j in range(D_MODEL // LANES):
          sl = pl.ds(j * LANES, LANES)
          scl_v[b, r, sl] = raw_v[b, r, sl] * scale

      out_start(c, b)

      @pl.when(c + NBUF < n_chunks)
      def _():
        gather_start(c + NBUF, b)

    return carry

  lax.fori_loop(0, n_chunks // NBUF, group, 0)

  for b in range(NBUF):
    out_wait(b)


def kernel(x, table):
  bsz, seq = x.shape
  vocab, d = table.shape
  assert d == D_MODEL
  total = bsz * seq
  assert total % (NUM_WORKERS * CHUNK) == 0
  n_chunks = total // (NUM_WORKERS * CHUNK)
  assert n_chunks % NBUF == 0

  xw = x.reshape(NUM_WORKERS, n_chunks, CHUNK).astype(jnp.int32)
  scale = jnp.float32(math.sqrt(d))

  mesh = plsc.VectorSubcoreMesh(
      core_axis_name="c", subcore_axis_name="s",
      num_cores=NUM_CORES, num_subcores=NUM_SUBCORES)

  emb = pl.kernel(
      functools.partial(_emb_body, n_chunks, scale),
      out_type=jax.ShapeDtypeStruct((total, d), jnp.float32),
      mesh=mesh,
      compiler_params=pltpu.CompilerParams(use_tc_tiling_on_sc=False),
      scratch_types=[
          pltpu.VMEM((n_chunks, CHUNK), jnp.int32),
          pltpu.VMEM((NBUF, CHUNK, d), jnp.float32),
          pltpu.VMEM((NBUF, CHUNK, d), jnp.float32),
          pltpu.SemaphoreType.DMA((NBUF,)),
          pltpu.SemaphoreType.DMA((NBUF,)),
      ],
  )(xw, table)

  return emb.reshape(bsz, seq, d)
